# baseline (speedup 1.0000x reference)
import jax
import jax.numpy as jnp
from jax.experimental import pallas as pl
from jax.experimental.pallas import tpu as pltpu

_TILE = 1024
_NTOK = 8192
_HALF = _NTOK // 2
_NSTEP = _HALF // _TILE


def _softmax(logits, b):
    logits = logits + b
    m = jnp.max(logits, axis=-1, keepdims=True)
    e = jnp.exp(logits - m)
    s = jnp.sum(e, axis=-1, keepdims=True)
    return e / s


def _gating_kernel(x1_ref, x2_ref, w_ref, b_ref, o1_ref, o2_ref):
    dn = (((1,), (1,)), ((), ()))
    l1 = jax.lax.dot_general(x1_ref[...], w_ref[...], dimension_numbers=dn,
                             preferred_element_type=jnp.float32)
    o1_ref[...] = _softmax(l1, b_ref[...])
    l2 = jax.lax.dot_general(x2_ref[...], w_ref[...], dimension_numbers=dn,
                             preferred_element_type=jnp.float32)
    o2_ref[...] = _softmax(l2, b_ref[...])


def kernel(x, W, b):
    n_tokens, input_dim = x.shape
    num_experts = W.shape[0]
    b2 = b.reshape(1, num_experts)
    out = pl.pallas_call(
        _gating_kernel,
        grid=(_NSTEP,),
        in_specs=[
            pl.BlockSpec((_TILE, input_dim), lambda i: (i, 0)),
            pl.BlockSpec((_TILE, input_dim), lambda i: (i + _NSTEP, 0)),
            pl.BlockSpec((num_experts, input_dim), lambda i: (0, 0)),
            pl.BlockSpec((1, num_experts), lambda i: (0, 0)),
        ],
        out_specs=[
            pl.BlockSpec((_TILE, num_experts), lambda i: (i, 0)),
            pl.BlockSpec((_TILE, num_experts), lambda i: (i, 0)),
        ],
        out_shape=[
            jax.ShapeDtypeStruct((_HALF, num_experts), jnp.float32),
            jax.ShapeDtypeStruct((_HALF, num_experts), jnp.float32),
        ],
        compiler_params=pltpu.CompilerParams(
            dimension_semantics=("arbitrary",),
        ),
    )(x, x, W, b2)
    return jnp.concatenate(out, axis=0)


# speedup vs baseline: 1.0525x; 1.0525x over previous
import jax
import jax.numpy as jnp
from jax.experimental import pallas as pl
from jax.experimental.pallas import tpu as pltpu

_TILE = 1024
_NTOK = 8192
_NBLK = _NTOK // _TILE
_NBUF = 4


def _gating_kernel(x_hbm, w_ref, b_ref, o_hbm):
    def inner(x_blk, o_blk):
        logits = jax.lax.dot_general(
            x_blk[...], w_ref[...],
            dimension_numbers=(((1,), (1,)), ((), ())),
            preferred_element_type=jnp.float32,
        )
        logits = logits + b_ref[...]
        m = jnp.max(logits, axis=-1, keepdims=True)
        e = jnp.exp(logits - m)
        s = jnp.sum(e, axis=-1, keepdims=True)
        o_blk[...] = e / s

    pipe = pltpu.emit_pipeline(
        inner,
        grid=(_NBLK,),
        in_specs=[
            pl.BlockSpec((_TILE, 2048), lambda i: (i, 0),
                         pipeline_mode=pl.Buffered(
                             buffer_count=_NBUF, use_lookahead=True)),
        ],
        out_specs=[
            pl.BlockSpec((_TILE, 64), lambda i: (i, 0)),
        ],
    )
    pipe(x_hbm, o_hbm)


def kernel(x, W, b):
    n_tokens, input_dim = x.shape
    num_experts = W.shape[0]
    b2 = b.reshape(1, num_experts)
    return pl.pallas_call(
        _gating_kernel,
        in_specs=[
            pl.BlockSpec(memory_space=pltpu.MemorySpace.HBM),
            pl.BlockSpec(memory_space=pltpu.MemorySpace.VMEM),
            pl.BlockSpec(memory_space=pltpu.MemorySpace.VMEM),
        ],
        out_specs=pl.BlockSpec(memory_space=pltpu.MemorySpace.HBM),
        out_shape=jax.ShapeDtypeStruct((n_tokens, num_experts), jnp.float32),
    )(x, W, b2)
